# baseline (speedup 1.0000x reference)
import functools

import jax
import jax.numpy as jnp
from jax import lax
from jax.experimental import pallas as pl
from jax.experimental.pallas import tpu as pltpu
from jax.experimental.pallas import tpu_sc as plsc

B, S, T, SP = 2, 8, 2048, 128
DT = 0.001
L = 16
NC, NS = 2, 16
NW = NC * NS
LG = (B * S * SP) // L
LG_PER_W = LG // NW
CH = 4
CK = 1024


def _min_tc_body(x_ref, y_ref, o_ref):
    m = jnp.minimum(jnp.min(x_ref[...]), jnp.min(y_ref[...]))

    @pl.when(pl.program_id(0) == 0)
    def _():
        o_ref[0, 0] = m

    @pl.when(pl.program_id(0) > 0)
    def _():
        o_ref[0, 0] = jnp.minimum(o_ref[0, 0], m)


def _global_min(x, y):
    x2 = x.reshape(4096, 1024)
    y2 = y.reshape(4096, 1024)
    grid = 8
    blk = 4096 // grid
    return pl.pallas_call(
        _min_tc_body,
        grid=(grid,),
        in_specs=[
            pl.BlockSpec((blk, 1024), lambda i: (i, 0)),
            pl.BlockSpec((blk, 1024), lambda i: (i, 0)),
        ],
        out_specs=pl.BlockSpec(memory_space=pltpu.SMEM),
        out_shape=jax.ShapeDtypeStruct((1, 1), jnp.float32),
    )(x2, y2)


def _sc_body(x_hbm, y_hbm, min_hbm, out_hbm, xstg, ystg, uf, vf, minv, outv):
    wid = lax.axis_index("s") * NC + lax.axis_index("c")
    pltpu.sync_copy(min_hbm.at[:], minv)
    m = minv[...]
    shift = jnp.where(m < 0.0, 1.1 * m, jnp.zeros_like(m))
    lane = lax.iota(jnp.int32, L)
    zf = jnp.zeros((L,), jnp.float32)
    zi = jnp.zeros((L,), jnp.int32)
    big = jnp.full((L,), 3.0e38, jnp.float32)
    c16 = jnp.full((L,), 16, jnp.int32)
    addr_cap = jnp.full((L,), (T - 1) * L, jnp.int32) + lane
    dt16 = jnp.float32(DT / L)

    acc_out = zf
    for k in range(LG_PER_W):
        g = wid * LG_PER_W + k
        b = g // (LG // B)
        rem = g % (LG // B)
        sidx = rem // (SP // L)
        sp0 = (rem % (SP // L)) * L

        cx = zf
        cy = zf
        for c in range(T // CK):
            pltpu.sync_copy(
                x_hbm.at[b, sidx, pl.ds(c * CK, CK), pl.ds(sp0, L)], xstg)
            pltpu.sync_copy(
                y_hbm.at[b, sidx, pl.ds(c * CK, CK), pl.ds(sp0, L)], ystg)
            base = c * CK * L

            def cs_body(i, carry, base=base):
                cx, cy = carry
                cx = cx + xstg[i]
                cy = cy + ystg[i]
                uf[pl.ds(base + i * L, L)] = cx
                vf[pl.ds(base + i * L, L)] = cy
                return cx, cy

            cx, cy = lax.fori_loop(0, CK, cs_body, (cx, cy), unroll=8)

        sx = cx - jnp.float32(T) * shift
        sy = cy - jnp.float32(T) * shift
        keep = jnp.logical_and(sx != 0.0, sy != 0.0)
        invdx = 1.0 / (sx + 1e-10)
        invdy = 1.0 / (sy + 1e-10)

        def af_body(i, carry):
            cnt = carry
            a = i * L
            uf[pl.ds(a, L)] = (uf[pl.ds(a, L)] - cnt * shift) * invdx
            vf[pl.ds(a, L)] = (vf[pl.ds(a, L)] - cnt * shift) * invdy
            return cnt + 1.0
        lax.fori_loop(0, T, af_body, jnp.full((L,), 1.0, jnp.float32),
                      unroll=8)
        uf[pl.ds(T * L, L)] = big
        vf[pl.ds(T * L, L)] = big

        st0 = []
        for c in range(CH):
            k0 = c * (2 * T // CH)
            if k0 == 0:
                st0.append((lane, lane, zf, zf))
                continue
            lo = jnp.full((L,), max(0, k0 - T), jnp.int32)
            hi = jnp.full((L,), min(k0, T), jnp.int32)

            def bs_body(r, carry, k0=k0):
                lo, hi = carry
                mid = lax.shift_right_logical(lo + hi, 1)
                um = plsc.load_gather(
                    uf, [jnp.minimum(mid, T - 1) * L + lane])
                vm = plsc.load_gather(
                    vf, [jnp.maximum(k0 - mid - 1, 0) * L + lane])
                f = um > vm
                active = lo < hi
                hi = jnp.where(jnp.logical_and(active, f), mid, hi)
                lo = jnp.where(jnp.logical_and(active, jnp.logical_not(f)),
                               mid + 1, lo)
                return lo, hi

            i0, _ = lax.fori_loop(0, 12, bs_body, (lo, hi))
            j0 = k0 - i0
            pu = plsc.load_gather(uf, [jnp.maximum(i0 - 1, 0) * L + lane])
            pv = plsc.load_gather(vf, [jnp.maximum(j0 - 1, 0) * L + lane])
            pu = jnp.where(i0 > 0, pu, -big)
            pv = jnp.where(j0 > 0, pv, -big)
            st0.append((i0 * L + lane, j0 * L + lane,
                        jnp.maximum(pu, pv), zf))

        def mg_body(step, carry):
            out = []
            for c in range(CH):
                ia, ja, prev, acc = carry[c]
                uc = plsc.load_gather(uf, [ia])
                vc = plsc.load_gather(vf, [ja])
                take_u = uc <= vc
                q = jnp.minimum(uc, vc)
                iac = jnp.minimum(ia, addr_cap)
                jac = jnp.minimum(ja, addr_cap)
                td = dt16 * (iac - jac).astype(jnp.float32)
                acc = acc + (q - prev) * (td * td)
                ti = jnp.where(take_u, c16, zi)
                out.append((ia + ti, ja + (c16 - ti), q, acc))
            return tuple(out)

        stf = lax.fori_loop(0, 2 * T // CH, mg_body, tuple(st0), unroll=2)
        w = stf[0][3]
        for c in range(1, CH):
            w = w + stf[c][3]
        acc_out = acc_out + jnp.where(keep, w, zf)

    outv[...] = acc_out
    pltpu.sync_copy(outv, out_hbm.at[wid])


def _sc_wasserstein(x, y, minvec):
    mesh = plsc.VectorSubcoreMesh(
        core_axis_name="c", subcore_axis_name="s",
        num_cores=NC, num_subcores=NS)
    f = functools.partial(
        pl.kernel,
        out_type=jax.ShapeDtypeStruct((NW, L), jnp.float32),
        mesh=mesh,
        scratch_types=[
            pltpu.VMEM((CK, L), jnp.float32),
            pltpu.VMEM((CK, L), jnp.float32),
            pltpu.VMEM((T * L + L,), jnp.float32),
            pltpu.VMEM((T * L + L,), jnp.float32),
            pltpu.VMEM((L,), jnp.float32),
            pltpu.VMEM((L,), jnp.float32),
        ],
        compiler_params=pltpu.CompilerParams(
            use_tc_tiling_on_sc=False, needs_layout_passes=False),
    )(_sc_body)
    return f(x, y, minvec)


def kernel(x, y):
    mn = _global_min(x, y)
    minvec = jnp.broadcast_to(mn.reshape(1), (L,))
    part = _sc_wasserstein(x, y, minvec)
    return part.reshape(B, NW // B * L).sum(axis=1)
